# baseline (speedup 1.0000x reference)
import functools

import jax
import jax.numpy as jnp
from jax import lax
from jax.experimental import pallas as pl
from jax.experimental.pallas import tpu as pltpu
from jax.experimental.pallas import tpu_sc as plsc

_NC = 2
_NS = 16
_NW = _NC * _NS
_LANE = 128


_HW = 128

_SC0_SHARE = 0.38


@functools.lru_cache(maxsize=None)
def _make_hist_kernel(ksteps, nz, rpt):
    mesh = plsc.VectorSubcoreMesh(core_axis_name="c", subcore_axis_name="s")

    @functools.partial(
        pl.kernel,
        mesh=mesh,
        out_type=jax.ShapeDtypeStruct((_NC, nz, _HW), jnp.float32),
        scratch_types=[
            pltpu.VMEM((ksteps, _LANE), jnp.int32),
            pltpu.VMEM((_LANE, _HW), jnp.float32),
            pltpu.VMEM_SHARED((nz, _HW), jnp.float32),
        ],
    )
    def hist_kernel(dst_hbm, ones_hbm, zeros_hbm, hist_hbm, dst_v, ones_v, hist_s):
        c = lax.axis_index("c")
        s = lax.axis_index("s")
        wid = c * _NS + s
        pltpu.sync_copy(zeros_hbm, hist_s.at[pl.ds(s * rpt, rpt)])
        pltpu.sync_copy(dst_hbm.at[wid], dst_v)
        pltpu.sync_copy(ones_hbm, ones_v)
        plsc.subcore_barrier()

        def body(j, carry):
            pltpu.sync_copy(ones_v, hist_s.at[dst_v.at[j]], add=True)
            return carry

        lax.fori_loop(0, ksteps, body, 0)
        plsc.subcore_barrier()
        pltpu.sync_copy(hist_s.at[pl.ds(s * rpt, rpt)],
                        hist_hbm.at[c, pl.ds(s * rpt, rpt)])

    return hist_kernel


@functools.lru_cache(maxsize=None)
def _make_msg_kernel(k0, k1, nz, d, rpt):
    kmax = max(k0, k1)
    mesh = plsc.VectorSubcoreMesh(core_axis_name="c", subcore_axis_name="s")

    @functools.partial(
        pl.kernel,
        mesh=mesh,
        out_type=jax.ShapeDtypeStruct((_NC, nz, d), jnp.float32),
        scratch_types=[
            pltpu.VMEM((kmax, _LANE), jnp.int32),
            pltpu.VMEM((kmax, _LANE), jnp.int32),
            pltpu.VMEM((_LANE, d), jnp.float32),
            pltpu.VMEM_SHARED((nz, d), jnp.float32),
            pltpu.SemaphoreType.DMA,
        ],
    )
    def msg_kernel(src_hbm, dst_hbm, y_hbm, zeros_hbm, acc_hbm,
                   src_v, dst_v, buf, acc_s, sem):
        c = lax.axis_index("c")
        s = lax.axis_index("s")
        wid = c * _NS + s
        ksteps = jnp.where(c == 0, k0, k1)
        pltpu.sync_copy(zeros_hbm, acc_s.at[pl.ds(s * rpt, rpt)])
        pltpu.sync_copy(src_hbm.at[wid], src_v)
        pltpu.sync_copy(dst_hbm.at[wid], dst_v)
        plsc.subcore_barrier()

        def body(j, carry):
            pltpu.async_copy(y_hbm.at[src_v.at[j]], buf, sem).wait()
            pltpu.sync_copy(buf, acc_s.at[dst_v.at[j]], add=True)
            return carry

        lax.fori_loop(0, ksteps, body, 0)
        plsc.subcore_barrier()
        pltpu.sync_copy(acc_s.at[pl.ds(s * rpt, rpt)],
                        acc_hbm.at[c, pl.ds(s * rpt, rpt)])

    return msg_kernel


def _prep_body(x_ref, w_ref, h0_ref, h1_ref, te_ref, tw_ref, ce_ref, cw_ref,
               b_ref, tb_ref, cb_ref, y_ref, dis_ref, brow_ref):
    deg = h0_ref[...] + h1_ref[...] + 1.0
    dis = lax.rsqrt(deg)
    xw = jnp.dot(x_ref[...], w_ref[...], preferred_element_type=jnp.float32)
    y_ref[...] = xw * dis
    dis_ref[...] = dis
    brow_ref[...] = (
        b_ref[...] + tb_ref[...] + cb_ref[...]
        + jnp.dot(te_ref[...], tw_ref[...], preferred_element_type=jnp.float32)
        + jnp.dot(ce_ref[...], cw_ref[...], preferred_element_type=jnp.float32)
    )


def _final_body(a0_ref, a1_ref, y_ref, dis_ref, brow_ref, o_ref):
    o_ref[...] = ((a0_ref[...] + a1_ref[...] + y_ref[...]) * dis_ref[...]
                  + brow_ref[...])


def kernel(x, t_emb, c_emb, edge_index, W, b, tW, tb, cW, cb):
    n, _ = x.shape
    d = W.shape[1]
    e = edge_index.shape[1]

    epw = -(-e // _NW)
    ksteps = -(-epw // _LANE)
    total = _NW * ksteps * _LANE
    rpt = -(-(n + 1) // _NS)
    rpt = -(-rpt // 64) * 64
    nz = rpt * _NS

    src = edge_index[0]
    dst = edge_index[1]
    pad = total - e
    def _spread(k):
        return jnp.arange(k, dtype=jnp.int32) % (nz - n) + n

    dst_u = jnp.concatenate([dst, _spread(pad)]).reshape(_NW, ksteps, _LANE)

    steps_total = -(-e // (_NS * _LANE))
    k0 = max(1, int(round(steps_total * _SC0_SHARE)))
    k1 = steps_total - k0 + 1
    kmax = max(k0, k1)
    e0 = _NS * k0 * _LANE
    cap = _NS * (k0 + k1) * _LANE
    padw = cap - e
    srcw = jnp.concatenate([src, jnp.zeros((padw,), jnp.int32)])
    dstw = jnp.concatenate([dst, _spread(padw)])

    def _wsplit(arr):
        a = arr[:e0].reshape(_NS, k0, _LANE)
        a = jnp.pad(a, ((0, 0), (0, kmax - k0), (0, 0)))
        bprt = arr[e0:].reshape(_NS, k1, _LANE)
        bprt = jnp.pad(bprt, ((0, 0), (0, kmax - k1), (0, 0)))
        return jnp.concatenate([a, bprt], axis=0)

    src_p = _wsplit(srcw)
    dst_p = _wsplit(dstw)

    ones_col = jnp.ones((_LANE, _HW), jnp.float32)
    zeros_hist = jnp.zeros((rpt, _HW), jnp.float32)
    zeros_acc = jnp.zeros((rpt, d), jnp.float32)

    hist = _make_hist_kernel(ksteps, nz, rpt)(dst_u, ones_col, zeros_hist)
    h0 = hist[0, :n, 0:1]
    h1 = hist[1, :n, 0:1]

    te = t_emb.reshape(1, -1)
    ce = c_emb.reshape(1, -1)
    b2 = b.reshape(1, -1)
    tb2 = tb.reshape(1, -1)
    cb2 = cb.reshape(1, -1)

    y, dis, brow = pl.pallas_call(
        _prep_body,
        out_shape=[
            jax.ShapeDtypeStruct((n, d), jnp.float32),
            jax.ShapeDtypeStruct((n, 1), jnp.float32),
            jax.ShapeDtypeStruct((1, d), jnp.float32),
        ],
    )(x, W, h0, h1, te, tW, ce, cW, b2, tb2, cb2)

    acc = _make_msg_kernel(k0, k1, nz, d, rpt)(src_p, dst_p, y, zeros_acc)
    a0 = acc[0, :n]
    a1 = acc[1, :n]

    br = 2000 if n % 2000 == 0 else n
    grid = (n // br,)
    out = pl.pallas_call(
        _final_body,
        grid=grid,
        in_specs=[
            pl.BlockSpec((br, d), lambda i: (i, 0)),
            pl.BlockSpec((br, d), lambda i: (i, 0)),
            pl.BlockSpec((br, d), lambda i: (i, 0)),
            pl.BlockSpec((br, 1), lambda i: (i, 0)),
            pl.BlockSpec((1, d), lambda i: (0, 0)),
        ],
        out_specs=pl.BlockSpec((br, d), lambda i: (i, 0)),
        out_shape=jax.ShapeDtypeStruct((n, d), jnp.float32),
    )(a0, a1, y, dis, brow)
    return out


# speedup vs baseline: 1.1341x; 1.1341x over previous
import functools

import jax
import jax.numpy as jnp
from jax import lax
from jax.experimental import pallas as pl
from jax.experimental.pallas import tpu as pltpu
from jax.experimental.pallas import tpu_sc as plsc

_NC = 2
_NS = 16
_NW = _NC * _NS
_LANE = 128


_HW = 128

_SC0_SHARE = 0.62


@functools.lru_cache(maxsize=None)
def _make_hist_kernel(ksteps, nz, rpt):
    mesh = plsc.VectorSubcoreMesh(core_axis_name="c", subcore_axis_name="s")

    @functools.partial(
        pl.kernel,
        mesh=mesh,
        out_type=jax.ShapeDtypeStruct((_NC, nz, _HW), jnp.float32),
        scratch_types=[
            pltpu.VMEM((ksteps, _LANE), jnp.int32),
            pltpu.VMEM((_LANE, _HW), jnp.float32),
            pltpu.VMEM_SHARED((nz, _HW), jnp.float32),
        ],
    )
    def hist_kernel(dst_hbm, ones_hbm, zeros_hbm, hist_hbm, dst_v, ones_v, hist_s):
        c = lax.axis_index("c")
        s = lax.axis_index("s")
        wid = c * _NS + s
        pltpu.sync_copy(zeros_hbm, hist_s.at[pl.ds(s * rpt, rpt)])
        pltpu.sync_copy(dst_hbm.at[wid], dst_v)
        pltpu.sync_copy(ones_hbm, ones_v)
        plsc.subcore_barrier()

        def body(j, carry):
            pltpu.sync_copy(ones_v, hist_s.at[dst_v.at[j]], add=True)
            return carry

        lax.fori_loop(0, ksteps, body, 0)
        plsc.subcore_barrier()
        pltpu.sync_copy(hist_s.at[pl.ds(s * rpt, rpt)],
                        hist_hbm.at[c, pl.ds(s * rpt, rpt)])

    return hist_kernel


@functools.lru_cache(maxsize=None)
def _make_msg_kernel(k0, k1, nz, d, rpt):
    kmax = max(k0, k1)
    mesh = plsc.VectorSubcoreMesh(core_axis_name="c", subcore_axis_name="s")

    @functools.partial(
        pl.kernel,
        mesh=mesh,
        out_type=jax.ShapeDtypeStruct((_NC, nz, d), jnp.float32),
        scratch_types=[
            pltpu.VMEM((kmax, _LANE), jnp.int32),
            pltpu.VMEM((kmax, _LANE), jnp.int32),
            pltpu.VMEM((_LANE, d), jnp.float32),
            pltpu.VMEM_SHARED((nz, d), jnp.float32),
            pltpu.SemaphoreType.DMA,
        ],
    )
    def msg_kernel(src_hbm, dst_hbm, y_hbm, zeros_hbm, acc_hbm,
                   src_v, dst_v, buf, acc_s, sem):
        c = lax.axis_index("c")
        s = lax.axis_index("s")
        wid = c * _NS + s
        ksteps = jnp.where(c == 0, k0, k1)
        pltpu.sync_copy(zeros_hbm, acc_s.at[pl.ds(s * rpt, rpt)])
        pltpu.sync_copy(src_hbm.at[wid], src_v)
        pltpu.sync_copy(dst_hbm.at[wid], dst_v)
        plsc.subcore_barrier()

        def body(j, carry):
            pltpu.async_copy(y_hbm.at[src_v.at[j]], buf, sem).wait()
            pltpu.sync_copy(buf, acc_s.at[dst_v.at[j]], add=True)
            return carry

        lax.fori_loop(0, ksteps, body, 0)
        plsc.subcore_barrier()
        pltpu.sync_copy(acc_s.at[pl.ds(s * rpt, rpt)],
                        acc_hbm.at[c, pl.ds(s * rpt, rpt)])

    return msg_kernel


def _prep_body(x_ref, w_ref, h0_ref, h1_ref, te_ref, tw_ref, ce_ref, cw_ref,
               b_ref, tb_ref, cb_ref, y_ref, dis_ref, brow_ref):
    deg = h0_ref[...] + h1_ref[...] + 1.0
    dis = lax.rsqrt(deg)
    xw = jnp.dot(x_ref[...], w_ref[...], preferred_element_type=jnp.float32)
    y_ref[...] = xw * dis
    dis_ref[...] = dis
    brow_ref[...] = (
        b_ref[...] + tb_ref[...] + cb_ref[...]
        + jnp.dot(te_ref[...], tw_ref[...], preferred_element_type=jnp.float32)
        + jnp.dot(ce_ref[...], cw_ref[...], preferred_element_type=jnp.float32)
    )


def _final_body(a0_ref, a1_ref, y_ref, dis_ref, brow_ref, o_ref):
    o_ref[...] = ((a0_ref[...] + a1_ref[...] + y_ref[...]) * dis_ref[...]
                  + brow_ref[...])


def kernel(x, t_emb, c_emb, edge_index, W, b, tW, tb, cW, cb):
    n, _ = x.shape
    d = W.shape[1]
    e = edge_index.shape[1]

    epw = -(-e // _NW)
    ksteps = -(-epw // _LANE)
    total = _NW * ksteps * _LANE
    rpt = -(-(n + 1) // _NS)
    rpt = -(-rpt // 64) * 64
    nz = rpt * _NS

    src = edge_index[0]
    dst = edge_index[1]
    pad = total - e
    def _spread(k):
        return jnp.arange(k, dtype=jnp.int32) % (nz - n) + n

    dst_u = jnp.concatenate([dst, _spread(pad)]).reshape(_NW, ksteps, _LANE)

    steps_total = -(-e // (_NS * _LANE))
    k0 = max(1, int(round(steps_total * _SC0_SHARE)))
    k1 = steps_total - k0 + 1
    kmax = max(k0, k1)
    e0 = _NS * k0 * _LANE
    cap = _NS * (k0 + k1) * _LANE
    padw = cap - e
    srcw = jnp.concatenate([src, jnp.zeros((padw,), jnp.int32)])
    dstw = jnp.concatenate([dst, _spread(padw)])

    def _wsplit(arr):
        a = arr[:e0].reshape(_NS, k0, _LANE)
        a = jnp.pad(a, ((0, 0), (0, kmax - k0), (0, 0)))
        bprt = arr[e0:].reshape(_NS, k1, _LANE)
        bprt = jnp.pad(bprt, ((0, 0), (0, kmax - k1), (0, 0)))
        return jnp.concatenate([a, bprt], axis=0)

    src_p = _wsplit(srcw)
    dst_p = _wsplit(dstw)

    ones_col = jnp.ones((_LANE, _HW), jnp.float32)
    zeros_hist = jnp.zeros((rpt, _HW), jnp.float32)
    zeros_acc = jnp.zeros((rpt, d), jnp.float32)

    hist = _make_hist_kernel(ksteps, nz, rpt)(dst_u, ones_col, zeros_hist)
    h0 = hist[0, :n, 0:1]
    h1 = hist[1, :n, 0:1]

    te = t_emb.reshape(1, -1)
    ce = c_emb.reshape(1, -1)
    b2 = b.reshape(1, -1)
    tb2 = tb.reshape(1, -1)
    cb2 = cb.reshape(1, -1)

    y, dis, brow = pl.pallas_call(
        _prep_body,
        out_shape=[
            jax.ShapeDtypeStruct((n, d), jnp.float32),
            jax.ShapeDtypeStruct((n, 1), jnp.float32),
            jax.ShapeDtypeStruct((1, d), jnp.float32),
        ],
    )(x, W, h0, h1, te, tW, ce, cW, b2, tb2, cb2)

    acc = _make_msg_kernel(k0, k1, nz, d, rpt)(src_p, dst_p, y, zeros_acc)
    a0 = acc[0, :n]
    a1 = acc[1, :n]

    br = 2000 if n % 2000 == 0 else n
    grid = (n // br,)
    out = pl.pallas_call(
        _final_body,
        grid=grid,
        in_specs=[
            pl.BlockSpec((br, d), lambda i: (i, 0)),
            pl.BlockSpec((br, d), lambda i: (i, 0)),
            pl.BlockSpec((br, d), lambda i: (i, 0)),
            pl.BlockSpec((br, 1), lambda i: (i, 0)),
            pl.BlockSpec((1, d), lambda i: (0, 0)),
        ],
        out_specs=pl.BlockSpec((br, d), lambda i: (i, 0)),
        out_shape=jax.ShapeDtypeStruct((n, d), jnp.float32),
    )(a0, a1, y, dis, brow)
    return out


# speedup vs baseline: 1.1480x; 1.0122x over previous
import functools

import jax
import jax.numpy as jnp
from jax import lax
from jax.experimental import pallas as pl
from jax.experimental.pallas import tpu as pltpu
from jax.experimental.pallas import tpu_sc as plsc

_NC = 2
_NS = 16
_NW = _NC * _NS
_LANE = 128


_HW = 128

_SC0_SHARE = 0.62


@functools.lru_cache(maxsize=None)
def _make_hist_kernel(ksteps, nz, rpt):
    mesh = plsc.VectorSubcoreMesh(core_axis_name="c", subcore_axis_name="s")

    @functools.partial(
        pl.kernel,
        mesh=mesh,
        out_type=jax.ShapeDtypeStruct((_NC, nz, _HW), jnp.float32),
        scratch_types=[
            pltpu.VMEM((ksteps, _LANE), jnp.int32),
            pltpu.VMEM((_LANE, _HW), jnp.float32),
            pltpu.VMEM_SHARED((nz, _HW), jnp.float32),
        ],
    )
    def hist_kernel(dst_hbm, ones_hbm, zeros_hbm, hist_hbm, dst_v, ones_v, hist_s):
        c = lax.axis_index("c")
        s = lax.axis_index("s")
        wid = c * _NS + s
        pltpu.sync_copy(zeros_hbm, hist_s.at[pl.ds(s * rpt, rpt)])
        pltpu.sync_copy(dst_hbm.at[wid], dst_v)
        pltpu.sync_copy(ones_hbm, ones_v)
        plsc.subcore_barrier()

        def body(j, carry):
            pltpu.sync_copy(ones_v, hist_s.at[dst_v.at[j]], add=True)
            return carry

        lax.fori_loop(0, ksteps, body, 0)
        plsc.subcore_barrier()
        pltpu.sync_copy(hist_s.at[pl.ds(s * rpt, rpt)],
                        hist_hbm.at[c, pl.ds(s * rpt, rpt)])

    return hist_kernel


@functools.lru_cache(maxsize=None)
def _make_msg_kernel(k0, k1, nz, d, rpt):
    kmax = max(k0, k1)
    mesh = plsc.VectorSubcoreMesh(core_axis_name="c", subcore_axis_name="s")

    @functools.partial(
        pl.kernel,
        mesh=mesh,
        out_type=jax.ShapeDtypeStruct((_NC, nz, d), jnp.float32),
        scratch_types=[
            pltpu.VMEM((kmax, _LANE), jnp.int32),
            pltpu.VMEM((kmax, _LANE), jnp.int32),
            pltpu.VMEM((_LANE, d), jnp.float32),
            pltpu.VMEM_SHARED((nz, d), jnp.float32),
            pltpu.SemaphoreType.DMA,
        ],
    )
    def msg_kernel(src_hbm, dst_hbm, y_hbm, zeros_hbm, acc_hbm,
                   src_v, dst_v, buf, acc_s, sem):
        c = lax.axis_index("c")
        s = lax.axis_index("s")
        wid = c * _NS + s
        ksteps = jnp.where(c == 0, k0, k1)
        pltpu.sync_copy(zeros_hbm, acc_s.at[pl.ds(s * rpt, rpt)])
        pltpu.sync_copy(src_hbm.at[wid], src_v)
        pltpu.sync_copy(dst_hbm.at[wid], dst_v)
        plsc.subcore_barrier()

        def body(j, carry):
            pltpu.async_copy(y_hbm.at[src_v.at[j]], buf, sem).wait()
            pltpu.sync_copy(buf, acc_s.at[dst_v.at[j]], add=True)
            return carry

        lax.fori_loop(0, ksteps, body, 0)
        plsc.subcore_barrier()
        pltpu.sync_copy(acc_s.at[pl.ds(s * rpt, rpt)],
                        acc_hbm.at[c, pl.ds(s * rpt, rpt)])

    return msg_kernel


def _prep_body(x_ref, w_ref, h0_ref, h1_ref, te_ref, tw_ref, ce_ref, cw_ref,
               b_ref, tb_ref, cb_ref, y_ref, dis_ref, brow_ref):
    deg = h0_ref[...] + h1_ref[...] + 1.0
    dis = lax.rsqrt(deg)
    xw = jnp.dot(x_ref[...], w_ref[...], preferred_element_type=jnp.float32)
    y_ref[...] = xw * dis
    dis_ref[...] = dis
    brow_ref[...] = (
        b_ref[...] + tb_ref[...] + cb_ref[...]
        + jnp.dot(te_ref[...], tw_ref[...], preferred_element_type=jnp.float32)
        + jnp.dot(ce_ref[...], cw_ref[...], preferred_element_type=jnp.float32)
    )


def _final_body(a0_ref, a1_ref, y_ref, dis_ref, brow_ref, o_ref):
    o_ref[...] = ((a0_ref[0] + a1_ref[0] + y_ref[...]) * dis_ref[...]
                  + brow_ref[...])


def kernel(x, t_emb, c_emb, edge_index, W, b, tW, tb, cW, cb):
    n, _ = x.shape
    d = W.shape[1]
    e = edge_index.shape[1]

    epw = -(-e // _NW)
    ksteps = -(-epw // _LANE)
    total = _NW * ksteps * _LANE
    rpt = -(-(n + 1) // _NS)
    rpt = -(-rpt // 64) * 64
    nz = rpt * _NS

    src = edge_index[0]
    dst = edge_index[1]
    pad = total - e
    def _spread(k):
        return jnp.arange(k, dtype=jnp.int32) % (nz - n) + n

    dst_u = jnp.concatenate([dst, _spread(pad)]).reshape(_NW, ksteps, _LANE)

    steps_total = -(-e // (_NS * _LANE))
    k0 = max(1, int(round(steps_total * _SC0_SHARE)))
    k1 = steps_total - k0 + 1
    kmax = max(k0, k1)
    e0 = _NS * k0 * _LANE
    cap = _NS * (k0 + k1) * _LANE
    padw = cap - e
    srcw = jnp.concatenate([src, jnp.zeros((padw,), jnp.int32)])
    dstw = jnp.concatenate([dst, _spread(padw)])

    def _wsplit(arr):
        a = arr[:e0].reshape(_NS, k0, _LANE)
        a = jnp.pad(a, ((0, 0), (0, kmax - k0), (0, 0)))
        bprt = arr[e0:].reshape(_NS, k1, _LANE)
        bprt = jnp.pad(bprt, ((0, 0), (0, kmax - k1), (0, 0)))
        return jnp.concatenate([a, bprt], axis=0)

    src_p = _wsplit(srcw)
    dst_p = _wsplit(dstw)

    ones_col = jnp.ones((_LANE, _HW), jnp.float32)
    zeros_hist = jnp.zeros((rpt, _HW), jnp.float32)
    zeros_acc = jnp.zeros((rpt, d), jnp.float32)

    hist = _make_hist_kernel(ksteps, nz, rpt)(dst_u, ones_col, zeros_hist)
    h0 = hist[0, :n, 0:1]
    h1 = hist[1, :n, 0:1]

    te = t_emb.reshape(1, -1)
    ce = c_emb.reshape(1, -1)
    b2 = b.reshape(1, -1)
    tb2 = tb.reshape(1, -1)
    cb2 = cb.reshape(1, -1)

    y, dis, brow = pl.pallas_call(
        _prep_body,
        out_shape=[
            jax.ShapeDtypeStruct((n, d), jnp.float32),
            jax.ShapeDtypeStruct((n, 1), jnp.float32),
            jax.ShapeDtypeStruct((1, d), jnp.float32),
        ],
    )(x, W, h0, h1, te, tW, ce, cW, b2, tb2, cb2)

    acc = _make_msg_kernel(k0, k1, nz, d, rpt)(src_p, dst_p, y, zeros_acc)

    br = 2000 if n % 2000 == 0 else n
    grid = (n // br,)
    out = pl.pallas_call(
        _final_body,
        grid=grid,
        in_specs=[
            pl.BlockSpec((1, br, d), lambda i: (0, i, 0)),
            pl.BlockSpec((1, br, d), lambda i: (1, i, 0)),
            pl.BlockSpec((br, d), lambda i: (i, 0)),
            pl.BlockSpec((br, 1), lambda i: (i, 0)),
            pl.BlockSpec((1, d), lambda i: (0, 0)),
        ],
        out_specs=pl.BlockSpec((br, d), lambda i: (i, 0)),
        out_shape=jax.ShapeDtypeStruct((n, d), jnp.float32),
    )(acc, acc, y, dis, brow)
    return out
